# baseline (speedup 1.0000x reference)
import functools

import numpy as np
import jax
import jax.numpy as jnp
from jax import lax
from jax.experimental import pallas as pl
from jax.experimental.pallas import tpu as pltpu
from jax.experimental.pallas import tpu_sc as plsc

B = 8
NQ = 5000
NT = 1000
NM = 1000
NP = 5008
NV = NP // 16
NOUT = 1024
NBINS = 256
NJOBS = 2 * B

_I32_MIN = np.int32(-(2**31))


def _area_key(x1, y1, x2, y2):
    a = (x2 - x1) * (y2 - y1)
    a = jnp.where(a == 0.0, 0.0, a)
    u = lax.bitcast_convert_type(a, jnp.int32)
    s = u >> 31
    m = u ^ (s | _I32_MIN)
    return ~m


def _keys_body(pt_ref, tt_ref, out_ref):
    kp = _area_key(pt_ref[0], pt_ref[1], pt_ref[2], pt_ref[3])
    kt = _area_key(tt_ref[0], tt_ref[1], tt_ref[2], tt_ref[3])
    pad = jnp.int32(-1)
    rowp = jnp.concatenate(
        [kp, jnp.full((B, NP - NQ), pad, jnp.int32)], axis=1)
    rowt = jnp.concatenate(
        [kt, jnp.full((B, NP - NT), pad, jnp.int32)], axis=1)
    out_ref[...] = jnp.concatenate([rowp, rowt], axis=0)


_build_keys = pl.pallas_call(
    _keys_body,
    out_shape=jax.ShapeDtypeStruct((NJOBS, NP), jnp.int32),
)


def _sc_sort_body(keys_hbm, out_hbm, ka, kb, va, vb, hist, scrp, scr2):
    wid = lax.axis_index("s") * 2 + lax.axis_index("c")

    @pl.when(wid < NJOBS)
    def _():
        lane = lax.iota(jnp.int32, 16)
        ones = jnp.ones(16, jnp.int32)

        pltpu.sync_copy(keys_hbm.at[wid], ka)

        def init_v(i, c):
            va[pl.ds(i * 16, 16)] = lane + i * 16
            return c
        lax.fori_loop(0, NV, init_v, 0)

        scrp[pl.ds(0, 16)] = jnp.full((16,), -16, jnp.int32)

        buf_cycle = [(ka, va, kb, vb), (kb, vb, ka, va)] * 2
        for p, (ki, vi, ko, vo) in enumerate(buf_cycle):
            shift = 8 * p

            def clr(j, c):
                hist[pl.ds(j * 16, 16)] = jnp.zeros((16,), jnp.int32)
                return c
            lax.fori_loop(0, NBINS // 16, clr, 0)

            def hist_body(i, c):
                k = ki[pl.ds(i * 16, 16)]
                d = lax.shift_right_logical(k, shift) & 255
                plsc.addupdate_scatter(hist, [d], ones)
                return c
            lax.fori_loop(0, NV, hist_body, 0)

            def scan_body(j, carry):
                h = hist[pl.ds(j * 16, 16)]
                inc = plsc.cumsum(h)
                hist[pl.ds(j * 16, 16)] = inc - h + carry
                return carry + jnp.sum(h)
            lax.fori_loop(0, NBINS // 16, scan_body, jnp.int32(0))

            def scat_body(i, c):
                k = ki[pl.ds(i * 16, 16)]
                v = vi[pl.ds(i * 16, 16)]
                d = lax.shift_right_logical(k, shift) & 255
                key2 = d * 16 + lane
                sk, _ = plsc.sort_key_val(key2, lane)
                scrp[pl.ds(1, 16)] = sk
                prev = scrp[pl.ds(0, 16)]
                ds_d = sk >> 4
                prev_d = prev >> 4
                seg = plsc.cummax(jnp.where(ds_d != prev_d, lane, 0))
                r_sorted = lane - seg
                orig = sk & 15
                plsc.store_scatter(scr2, [orig], r_sorted)
                r = scr2[...]
                cur = plsc.load_gather(hist, [d])
                pos = cur + r
                plsc.store_scatter(ko, [pos], k)
                plsc.store_scatter(vo, [pos], v)
                plsc.addupdate_scatter(hist, [d], ones)
                return c
            lax.fori_loop(0, NV, scat_body, 0)

        pltpu.sync_copy(va.at[pl.ds(0, NOUT)], out_hbm.at[wid])


def _make_sc_sort(interpret=False, **mesh_kw):
    return pl.kernel(
        _sc_sort_body,
        out_type=jax.ShapeDtypeStruct((NJOBS, NOUT), jnp.int32),
        mesh=plsc.VectorSubcoreMesh(
            core_axis_name="c", subcore_axis_name="s", **mesh_kw),
        compiler_params=pltpu.CompilerParams(needs_layout_passes=False),
        interpret=interpret,
        scratch_types=[
            pltpu.VMEM((NP,), jnp.int32),
            pltpu.VMEM((NP,), jnp.int32),
            pltpu.VMEM((NP,), jnp.int32),
            pltpu.VMEM((NP,), jnp.int32),
            pltpu.VMEM((NBINS,), jnp.int32),
            pltpu.VMEM((32,), jnp.int32),
            pltpu.VMEM((16,), jnp.int32),
        ],
    )


@functools.cache
def _get_sc_sort():
    return _make_sc_sort()


def kernel(logits, pred_boxes, target_boxes, class_labels):
    del logits, class_labels
    pt = jnp.transpose(pred_boxes, (2, 0, 1))
    tt = jnp.transpose(target_boxes, (2, 0, 1))
    keys = _build_keys(pt, tt)
    out = _get_sc_sort()(keys)
    matched_pred = out[:B, :NM]
    matched_target = out[B:, :NM]
    return (matched_pred, matched_target)


# speedup vs baseline: 1.9484x; 1.9484x over previous
import functools

import numpy as np
import jax
import jax.numpy as jnp
from jax import lax
from jax.experimental import pallas as pl
from jax.experimental.pallas import tpu as pltpu
from jax.experimental.pallas import tpu_sc as plsc

B = 8
NQ = 5000
NT = 1000
NM = 1000
NP = 5008
NV = NP // 16
NOUT = 1024
NBINS = 256
NJOBS = 2 * B

_I32_MIN = np.int32(-(2**31))


def _area_key(x1, y1, x2, y2):
    a = (x2 - x1) * (y2 - y1)
    a = jnp.where(a == 0.0, 0.0, a)
    u = lax.bitcast_convert_type(a, jnp.int32)
    s = u >> 31
    m = u ^ (s | _I32_MIN)
    return ~m


def _keys_body(pt_ref, tt_ref, out_ref):
    kp = _area_key(pt_ref[0], pt_ref[1], pt_ref[2], pt_ref[3])
    kt = _area_key(tt_ref[0], tt_ref[1], tt_ref[2], tt_ref[3])
    pad = jnp.int32(-1)
    rowp = jnp.concatenate(
        [kp, jnp.full((B, NP - NQ), pad, jnp.int32)], axis=1)
    rowt = jnp.concatenate(
        [kt, jnp.full((B, NP - NT), pad, jnp.int32)], axis=1)
    out_ref[...] = jnp.concatenate([rowp, rowt], axis=0)


_build_keys = pl.pallas_call(
    _keys_body,
    out_shape=jax.ShapeDtypeStruct((NJOBS, NP), jnp.int32),
)


def _sc_sort_body(keys_hbm, out_hbm, kin, ka, kb, va, vb, hist, hist2, scrp, scr2):
    wid = lax.axis_index("s") * 2 + lax.axis_index("c")

    @pl.when(wid < NJOBS)
    def _():
        lane = lax.iota(jnp.int32, 16)
        ones = jnp.ones(16, jnp.int32)
        zeros16 = jnp.zeros((16,), jnp.int32)

        pltpu.sync_copy(keys_hbm.at[wid], kin)

        scrp[pl.ds(0, 16)] = jnp.full((16,), -16, jnp.int32)

        def clr(h):
            def body(j, c):
                h[pl.ds(j * 16, 16)] = zeros16
                return c
            lax.fori_loop(0, NBINS // 16, body, 0)

        clr(hist)

        def hist_a(i, c):
            k = kin[pl.ds(i * 16, 16)]
            d = lax.shift_right_logical(k, 24) & 255
            plsc.addupdate_scatter(hist, [d], ones)
            return c
        lax.fori_loop(0, NV, hist_a, 0)

        def scan_a(j, carry):
            tot, c = carry
            h = hist[pl.ds(j * 16, 16)]
            inc = plsc.cumsum(h) + tot
            c = c + jnp.sum(jnp.where(inc < NM, 1, 0))
            return (tot + jnp.sum(h), c)
        _, cut = lax.fori_loop(
            0, NBINS // 16, scan_a, (jnp.int32(0), jnp.int32(0)))

        clr(hist)

        def compact(i, off):
            k = kin[pl.ds(i * 16, 16)]
            d = lax.shift_right_logical(k, 24) & 255
            m = d <= cut
            plsc.store_compressed(kb.at[pl.ds(off, 16)], k, mask=m)
            plsc.store_compressed(vb.at[pl.ds(off, 16)], lane + i * 16, mask=m)
            d0 = k & 255
            plsc.addupdate_scatter(hist, [d0], ones, mask=m)
            return off + jnp.sum(jnp.where(m, 1, 0))
        off = lax.fori_loop(0, NV, compact, jnp.int32(0))

        kb[pl.ds(off, 16)] = jnp.full((16,), -1, jnp.int32)
        vb[pl.ds(off, 16)] = lane + NP
        plsc.addupdate_scatter(hist, [jnp.full((16,), 255, jnp.int32)], ones)
        t2 = off // 16 + 1

        bufs = [
            (kb, vb, ka, va, hist, hist2),
            (ka, va, kb, vb, hist2, hist),
            (kb, vb, ka, va, hist, hist2),
            (ka, va, kb, vb, hist2, hist),
        ]
        for p, (ki, vi, ko, vo, hc, hn) in enumerate(bufs):
            shift = 8 * p
            if p < 3:
                clr(hn)

            def scan_c(j, carry):
                h = hc[pl.ds(j * 16, 16)]
                inc = plsc.cumsum(h)
                hc[pl.ds(j * 16, 16)] = inc - h + carry
                return carry + jnp.sum(h)
            lax.fori_loop(0, NBINS // 16, scan_c, jnp.int32(0))

            def scat_body(i, c):
                k = ki[pl.ds(i * 16, 16)]
                v = vi[pl.ds(i * 16, 16)]
                d = lax.shift_right_logical(k, shift) & 255
                key2 = d * 16 + lane
                sk, _ = plsc.sort_key_val(key2, lane)
                scrp[pl.ds(1, 16)] = sk
                prev = scrp[pl.ds(0, 16)]
                seg = plsc.cummax(jnp.where((sk >> 4) != (prev >> 4), lane, 0))
                r_sorted = lane - seg
                plsc.store_scatter(scr2, [sk & 15], r_sorted)
                r = scr2[...]
                cur = plsc.load_gather(hc, [d])
                pos = cur + r
                plsc.store_scatter(ko, [pos], k)
                plsc.store_scatter(vo, [pos], v)
                plsc.addupdate_scatter(hc, [d], ones)
                if p < 3:
                    dn = lax.shift_right_logical(k, shift + 8) & 255
                    plsc.addupdate_scatter(hn, [dn], ones)
                return c
            lax.fori_loop(0, t2, scat_body, 0)

        pltpu.sync_copy(vb.at[pl.ds(0, NOUT)], out_hbm.at[wid])


def _make_sc_sort(interpret=False, **mesh_kw):
    return pl.kernel(
        _sc_sort_body,
        out_type=jax.ShapeDtypeStruct((NJOBS, NOUT), jnp.int32),
        mesh=plsc.VectorSubcoreMesh(
            core_axis_name="c", subcore_axis_name="s", **mesh_kw),
        compiler_params=pltpu.CompilerParams(needs_layout_passes=False),
        interpret=interpret,
        scratch_types=[
            pltpu.VMEM((NP,), jnp.int32),
            pltpu.VMEM((NP + 32,), jnp.int32),
            pltpu.VMEM((NP + 32,), jnp.int32),
            pltpu.VMEM((NP + 32,), jnp.int32),
            pltpu.VMEM((NP + 32,), jnp.int32),
            pltpu.VMEM((NBINS,), jnp.int32),
            pltpu.VMEM((NBINS,), jnp.int32),
            pltpu.VMEM((32,), jnp.int32),
            pltpu.VMEM((16,), jnp.int32),
        ],
    )


@functools.cache
def _get_sc_sort():
    return _make_sc_sort()


def kernel(logits, pred_boxes, target_boxes, class_labels):
    del logits, class_labels
    pt = jnp.transpose(pred_boxes, (2, 0, 1))
    tt = jnp.transpose(target_boxes, (2, 0, 1))
    keys = _build_keys(pt, tt)
    out = _get_sc_sort()(keys)
    matched_pred = out[:B, :NM]
    matched_target = out[B:, :NM]
    return (matched_pred, matched_target)


# speedup vs baseline: 1.9986x; 1.0257x over previous
import functools

import numpy as np
import jax
import jax.numpy as jnp
from jax import lax
from jax.experimental import pallas as pl
from jax.experimental.pallas import tpu as pltpu
from jax.experimental.pallas import tpu_sc as plsc

B = 8
NQ = 5000
NT = 1000
NM = 1000
NP = 5008
NV = NP // 16
NOUT = 1024
NBINS = 256
NJOBS = 2 * B

_I32_MIN = np.int32(-(2**31))


def _area_key(x1, y1, x2, y2):
    a = (x2 - x1) * (y2 - y1)
    a = jnp.where(a == 0.0, 0.0, a)
    u = lax.bitcast_convert_type(a, jnp.int32)
    s = u >> 31
    m = u ^ (s | _I32_MIN)
    return ~m


def _keys_body(pt_ref, tt_ref, out_ref):
    kp = _area_key(pt_ref[0], pt_ref[1], pt_ref[2], pt_ref[3])
    kt = _area_key(tt_ref[0], tt_ref[1], tt_ref[2], tt_ref[3])
    pad = jnp.int32(-1)
    rowp = jnp.concatenate(
        [kp, jnp.full((B, NP - NQ), pad, jnp.int32)], axis=1)
    rowt = jnp.concatenate(
        [kt, jnp.full((B, NP - NT), pad, jnp.int32)], axis=1)
    out_ref[...] = jnp.concatenate([rowp, rowt], axis=0)


_build_keys = pl.pallas_call(
    _keys_body,
    out_shape=jax.ShapeDtypeStruct((NJOBS, NP), jnp.int32),
)


def _sc_sort_body(keys_hbm, out_hbm, kin, ka, kb, va, vb, hist, hist2, scrp, scr2):
    wid = lax.axis_index("s")

    @pl.when(wid < NJOBS)
    def _():
        lane = lax.iota(jnp.int32, 16)
        ones = jnp.ones(16, jnp.int32)
        zeros16 = jnp.zeros((16,), jnp.int32)

        pltpu.sync_copy(keys_hbm.at[wid], kin)

        scrp[pl.ds(0, 16)] = jnp.full((16,), -16, jnp.int32)

        def clr(h):
            def body(j, c):
                h[pl.ds(j * 16, 16)] = zeros16
                return c
            lax.fori_loop(0, NBINS // 16, body, 0)

        clr(hist)

        def hist_a(i, c):
            k = kin[pl.ds(i * 16, 16)]
            d = lax.shift_right_logical(k, 24) & 255
            plsc.addupdate_scatter(hist, [d], ones)
            return c
        lax.fori_loop(0, NV, hist_a, 0)

        def scan_a(j, carry):
            tot, c = carry
            h = hist[pl.ds(j * 16, 16)]
            inc = plsc.cumsum(h) + tot
            c = c + jnp.sum(jnp.where(inc < NM, 1, 0))
            return (tot + jnp.sum(h), c)
        _, cut = lax.fori_loop(
            0, NBINS // 16, scan_a, (jnp.int32(0), jnp.int32(0)))

        clr(hist)

        def compact(i, off):
            k = kin[pl.ds(i * 16, 16)]
            d = lax.shift_right_logical(k, 24) & 255
            m = d <= cut
            plsc.store_compressed(kb.at[pl.ds(off, 16)], k, mask=m)
            plsc.store_compressed(vb.at[pl.ds(off, 16)], lane + i * 16, mask=m)
            d0 = k & 255
            plsc.addupdate_scatter(hist, [d0], ones, mask=m)
            return off + jnp.sum(jnp.where(m, 1, 0))
        off = lax.fori_loop(0, NV, compact, jnp.int32(0))

        kb[pl.ds(off, 16)] = jnp.full((16,), -1, jnp.int32)
        vb[pl.ds(off, 16)] = lane + NP
        plsc.addupdate_scatter(hist, [jnp.full((16,), 255, jnp.int32)], ones)
        t2 = off // 16 + 1

        bufs = [
            (kb, vb, ka, va, hist, hist2),
            (ka, va, kb, vb, hist2, hist),
            (kb, vb, ka, va, hist, hist2),
            (ka, va, kb, vb, hist2, hist),
        ]
        for p, (ki, vi, ko, vo, hc, hn) in enumerate(bufs):
            shift = 8 * p
            if p < 3:
                clr(hn)

            def scan_c(j, carry):
                h = hc[pl.ds(j * 16, 16)]
                inc = plsc.cumsum(h)
                hc[pl.ds(j * 16, 16)] = inc - h + carry
                return carry + jnp.sum(h)
            lax.fori_loop(0, NBINS // 16, scan_c, jnp.int32(0))

            def scat_body(i, c):
                k = ki[pl.ds(i * 16, 16)]
                v = vi[pl.ds(i * 16, 16)]
                d = lax.shift_right_logical(k, shift) & 255
                key2 = d * 16 + lane
                sk, _ = plsc.sort_key_val(key2, lane)
                scrp[pl.ds(1, 16)] = sk
                prev = scrp[pl.ds(0, 16)]
                seg = plsc.cummax(jnp.where((sk >> 4) != (prev >> 4), lane, 0))
                r_sorted = lane - seg
                plsc.store_scatter(scr2, [sk & 15], r_sorted)
                r = scr2[...]
                cur = plsc.load_gather(hc, [d])
                pos = cur + r
                plsc.store_scatter(ko, [pos], k)
                plsc.store_scatter(vo, [pos], v)
                plsc.addupdate_scatter(hc, [d], ones)
                if p < 3:
                    dn = lax.shift_right_logical(k, shift + 8) & 255
                    plsc.addupdate_scatter(hn, [dn], ones)
                return c
            lax.fori_loop(0, t2, scat_body, 0)

        pltpu.sync_copy(vb.at[pl.ds(0, NOUT)], out_hbm.at[wid])


def _make_sc_sort(interpret=False, **mesh_kw):
    return pl.kernel(
        _sc_sort_body,
        out_type=jax.ShapeDtypeStruct((NJOBS, NOUT), jnp.int32),
        mesh=plsc.VectorSubcoreMesh(
            core_axis_name="c", subcore_axis_name="s", num_cores=1,
            **mesh_kw),
        compiler_params=pltpu.CompilerParams(needs_layout_passes=False),
        interpret=interpret,
        scratch_types=[
            pltpu.VMEM((NP,), jnp.int32),
            pltpu.VMEM((NP + 32,), jnp.int32),
            pltpu.VMEM((NP + 32,), jnp.int32),
            pltpu.VMEM((NP + 32,), jnp.int32),
            pltpu.VMEM((NP + 32,), jnp.int32),
            pltpu.VMEM((NBINS,), jnp.int32),
            pltpu.VMEM((NBINS,), jnp.int32),
            pltpu.VMEM((32,), jnp.int32),
            pltpu.VMEM((16,), jnp.int32),
        ],
    )


@functools.cache
def _get_sc_sort():
    return _make_sc_sort()


def kernel(logits, pred_boxes, target_boxes, class_labels):
    del logits, class_labels
    pt = jnp.transpose(pred_boxes, (2, 0, 1))
    tt = jnp.transpose(target_boxes, (2, 0, 1))
    keys = _build_keys(pt, tt)
    out = _get_sc_sort()(keys)
    matched_pred = out[:B, :NM]
    matched_target = out[B:, :NM]
    return (matched_pred, matched_target)
